# baseline (speedup 1.0000x reference)
import functools

import jax
import jax.numpy as jnp
from jax import lax
from jax.experimental import pallas as pl
from jax.experimental.pallas import tpu as pltpu
from jax.experimental.pallas import tpu_sc as plsc

_DIM = 64
_VOCAB = 100000
_NW = 32
_MAGIC = 12582912.0
_MV = (127.0, 31.0, 7.0)

_N0 = 256 * _DIM // _NW
_N1 = 1792 * _DIM // _NW
_N2 = 14336 * _DIM // _NW
_N3 = 83616 * _DIM // _NW
_C3 = _N3 // 3

_O0 = 0
_O1 = 256 * _DIM
_O2 = 2048 * _DIM
_O3 = 16384 * _DIM

_mesh = functools.partial(
    pl.kernel,
    mesh=plsc.VectorSubcoreMesh(core_axis_name="c", subcore_axis_name="s"),
    compiler_params=pltpu.CompilerParams(use_tc_tiling_on_sc=False),
)

_mesh_tiled = functools.partial(
    pl.kernel,
    mesh=plsc.VectorSubcoreMesh(core_axis_name="c", subcore_axis_name="s"),
    compiler_params=pltpu.CompilerParams(use_tc_tiling_on_sc=True),
)


def _tile_id():
    return lax.axis_index("s") * 2 + lax.axis_index("c")


def _reduce_chunk(buf, nrows, unroll_rows, carry):
    iters = nrows // unroll_rows
    assert iters * unroll_rows == nrows

    def body(it, ms):
        ms = list(ms)
        base = it * unroll_rows
        for j in range(unroll_rows):
            for c in range(4):
                v = buf[base + j, pl.ds(c * 16, 16)]
                ms[c] = jnp.maximum(ms[c], jnp.abs(v))
        return tuple(ms)

    return lax.fori_loop(0, iters, body, carry)


_R1 = _N1 // _DIM
_R2 = _N2 // _DIM
_R3 = _N3 // _DIM
_CR3 = _R3 // 3
_R3T = 2640
_CR3T = _R3T // 6


def _sc_scales(w1, w2, w3):
    @functools.partial(
        _mesh_tiled,
        out_type=jax.ShapeDtypeStruct((_NW, 12, 16), jnp.float32),
        scratch_types=[
            pltpu.VMEM((_CR3T, _DIM), jnp.float32),
            pltpu.VMEM((_CR3T, _DIM), jnp.float32),
            pltpu.VMEM((12, 16), jnp.float32),
            pltpu.SemaphoreType.DMA,
        ],
    )
    def k(w1_hbm, w2_hbm, w3_hbm, out_hbm, buf0, buf1, acc, sem):
        tid = _tile_id()
        zero4 = (jnp.zeros(16, jnp.float32),) * 4
        bufs = (buf0, buf1)

        w3_base = jnp.minimum(tid * _R3T, 83616 - _R3T)

        phases = [(w1_hbm, tid * _R1, _R1, 7, 0),
                  (w2_hbm, tid * _R2, _R2 // 2, 8, 1),
                  (w2_hbm, tid * _R2 + _R2 // 2, _R2 // 2, 8, 1)]
        phases += [(w3_hbm, w3_base + kk * _CR3T, _CR3T, 8, 2)
                   for kk in range(6)]
        n = len(phases)

        def start(i):
            src, off, rows, _, _ = phases[i]
            return pltpu.async_copy(src.at[pl.ds(off, rows)],
                                    bufs[i % 2].at[pl.ds(0, rows)], sem)

        ms = [zero4, zero4, zero4]
        h = [None] * n
        h[0] = start(0)
        for i in range(n):
            if i + 1 < n:
                h[i + 1] = start(i + 1)
            h[i].wait()
            _, _, rows, unroll, slot = phases[i]
            ms[slot] = _reduce_chunk(bufs[i % 2], rows, unroll, ms[slot])

        for c in range(4):
            acc[0 + c, :] = ms[0][c]
            acc[4 + c, :] = ms[1][c]
            acc[8 + c, :] = ms[2][c]
        pltpu.sync_copy(acc, out_hbm.at[tid])

    return k(w1, w2, w3)


def _quant_chunk(buf, nrows, unroll_rows, pre, sc):
    iters = nrows // unroll_rows
    assert iters * unroll_rows == nrows

    def body(it, carry):
        base = it * unroll_rows
        for j in range(unroll_rows):
            for c in range(4):
                v = buf[base + j, pl.ds(c * 16, 16)]
                r = (v * pre[c] + _MAGIC) - _MAGIC
                buf[base + j, pl.ds(c * 16, 16)] = r * sc
        return carry

    lax.fori_loop(0, iters, body, 0)


def _scale_chunk(buf, nrows, svecs):
    def body(it, carry):
        for c in range(4):
            v = buf[it, pl.ds(c * 16, 16)]
            buf[it, pl.ds(c * 16, 16)] = v * svecs[c]
        return carry

    lax.fori_loop(0, nrows, body, 0)


def _sc_table(partials, w0, w1, w2, w3, s0, s1, s2, s3):
    @functools.partial(
        _mesh,
        out_type=jax.ShapeDtypeStruct((_VOCAB, _DIM), jnp.float32),
        scratch_types=[
            pltpu.VMEM((_CR3, _DIM), jnp.float32),
            pltpu.VMEM((_CR3, _DIM), jnp.float32),
            pltpu.VMEM((_NW, 12, 16), jnp.float32),
            pltpu.VMEM((4, 64), jnp.float32),
            pltpu.SemaphoreType.DMA,
            pltpu.SemaphoreType.DMA,
        ],
    )
    def k(p_hbm, w0, w1, w2, w3, s0, s1, s2, s3, out_hbm,
          buf0, buf1, pbuf, sbuf, sem_i, sem_o):
        tid = _tile_id()
        bufs = (buf0, buf1)

        r0 = 256 // _NW
        pre_h = [
            pltpu.async_copy(w0.at[pl.ds(tid * r0, r0)],
                             bufs[0].at[pl.ds(0, r0)], sem_i),
            pltpu.async_copy(w1.at[pl.ds(tid * _R1, _R1)],
                             bufs[1].at[pl.ds(0, _R1)], sem_i),
        ]
        pltpu.sync_copy(p_hbm, pbuf)
        for i, s in enumerate((s0, s1, s2, s3)):
            pltpu.sync_copy(s.at[0], sbuf.at[i])

        svecs = [[sbuf[t, pl.ds(c * 16, 16)] for c in range(4)]
                 for t in range(4)]
        scs = []
        for t in range(3):
            m = [pbuf[0, t * 4 + c, :] for c in range(4)]
            for i in range(1, _NW):
                for c in range(4):
                    m[c] = jnp.maximum(m[c], pbuf[i, t * 4 + c, :])
            mm = jnp.maximum(jnp.maximum(m[0] * jnp.abs(svecs[t + 1][0]),
                                         m[1] * jnp.abs(svecs[t + 1][1])),
                             jnp.maximum(m[2] * jnp.abs(svecs[t + 1][2]),
                                         m[3] * jnp.abs(svecs[t + 1][3])))
            mx = mm[0]
            for i in range(1, 16):
                mx = jnp.maximum(mx, mm[i])
            mb = jnp.zeros((16,), jnp.float32) + jnp.maximum(mx, 1e-8)
            sc = mb / _MV[t]
            inv = 1.0 / sc
            scs.append(([svecs[t + 1][c] * inv for c in range(4)], sc))

        phases = [(w0, tid * r0, r0, 0, tid * r0, 0),
                  (w1, tid * _R1, _R1, 7, 256 + tid * _R1, 1),
                  (w2, tid * _R2, _R2, 7, 2048 + tid * _R2, 2)]
        phases += [(w3, tid * _R3 + kk * _CR3, _CR3, 13,
                    16384 + tid * _R3 + kk * _CR3, 3) for kk in range(3)]
        n = len(phases)

        def start_in(i):
            src, off, rows = phases[i][:3]
            return pltpu.async_copy(src.at[pl.ds(off, rows)],
                                    bufs[i % 2].at[pl.ds(0, rows)], sem_i)

        h_in = [None] * n
        h_out = [None] * n
        h_in[0], h_in[1] = pre_h
        for i in range(n):
            if i + 1 < n:
                if i >= 1:
                    h_out[i - 1].wait()
                if i + 1 >= 2:
                    h_in[i + 1] = start_in(i + 1)
            h_in[i].wait()
            _, _, rows, unroll, out_off, tier = phases[i]
            if tier == 0:
                _scale_chunk(bufs[i % 2], rows, svecs[0])
            else:
                _quant_chunk(bufs[i % 2], rows, unroll, *scs[tier - 1])
            h_out[i] = pltpu.async_copy(
                bufs[i % 2].at[pl.ds(0, rows)],
                out_hbm.at[pl.ds(out_off, rows)], sem_o)
        h_out[n - 2].wait()
        h_out[n - 1].wait()

    return k(partials, w0, w1, w2, w3, s0, s1, s2, s3)


def _sc_gather(table, idx):
    B = idx.shape[0]
    b_per_w = B // _NW
    C = 800
    n_chunks = b_per_w // C

    @functools.partial(
        _mesh,
        out_type=jax.ShapeDtypeStruct((B, _DIM), jnp.float32),
        scratch_types=[
            pltpu.VMEM((n_chunks, C), jnp.int32),
            pltpu.VMEM((C, _DIM), jnp.float32),
            pltpu.VMEM((C, _DIM), jnp.float32),
            pltpu.SemaphoreType.DMA,
            pltpu.SemaphoreType.DMA,
            pltpu.SemaphoreType.DMA,
        ],
    )
    def k(table_hbm, idx_hbm, out_hbm, idx_v, rows0, rows1, sem_i, sem_g, sem_s):
        base = _tile_id() * b_per_w
        rows = (rows0, rows1)

        ih = [pltpu.async_copy(idx_hbm.at[pl.ds(base + c * C, C)],
                               idx_v.at[c], sem_i)
              for c in range(n_chunks)]
        for h in ih:
            h.wait()

        gh = [None] * n_chunks
        sh = [None] * n_chunks
        gh[0] = pltpu.async_copy(table_hbm.at[idx_v.at[0]], rows[0], sem_g)
        for c in range(n_chunks):
            if c + 1 < n_chunks:
                if c >= 1:
                    sh[c - 1].wait()
                gh[c + 1] = pltpu.async_copy(
                    table_hbm.at[idx_v.at[c + 1]], rows[(c + 1) % 2], sem_g)
            gh[c].wait()
            sh[c] = pltpu.async_copy(
                rows[c % 2], out_hbm.at[pl.ds(base + c * C, C)], sem_s)
        sh[n_chunks - 2].wait()
        sh[n_chunks - 1].wait()

    return k(table, idx)


def kernel(input_ids, w0, w1, w2, w3, s0, s1, s2, s3):
    partials = _sc_scales(w1, w2, w3)
    table = _sc_table(partials, w0, w1, w2, w3, s0, s1, s2, s3)
    idx = input_ids.reshape(-1).astype(jnp.int32)
    out = _sc_gather(table, idx)
    return out.reshape(input_ids.shape + (_DIM,))


# speedup vs baseline: 1.0609x; 1.0609x over previous
import functools

import jax
import jax.numpy as jnp
from jax import lax
from jax.experimental import pallas as pl
from jax.experimental.pallas import tpu as pltpu
from jax.experimental.pallas import tpu_sc as plsc

_DIM = 64
_VOCAB = 100000
_NW = 32
_MAGIC = 12582912.0
_MV = (127.0, 31.0, 7.0)

_N0 = 256 * _DIM // _NW
_N1 = 1792 * _DIM // _NW
_N2 = 14336 * _DIM // _NW
_N3 = 83616 * _DIM // _NW
_C3 = _N3 // 3

_O0 = 0
_O1 = 256 * _DIM
_O2 = 2048 * _DIM
_O3 = 16384 * _DIM

_mesh = functools.partial(
    pl.kernel,
    mesh=plsc.VectorSubcoreMesh(core_axis_name="c", subcore_axis_name="s"),
    compiler_params=pltpu.CompilerParams(use_tc_tiling_on_sc=False),
)

_mesh_tiled = functools.partial(
    pl.kernel,
    mesh=plsc.VectorSubcoreMesh(core_axis_name="c", subcore_axis_name="s"),
    compiler_params=pltpu.CompilerParams(use_tc_tiling_on_sc=True),
)


def _tile_id():
    return lax.axis_index("s") * 2 + lax.axis_index("c")


def _reduce_chunk(buf, nrows, unroll_rows, carry):
    iters = nrows // unroll_rows
    assert iters * unroll_rows == nrows

    def body(it, ms):
        ms = list(ms)
        base = it * unroll_rows
        for j in range(unroll_rows):
            for c in range(4):
                v = buf[base + j, pl.ds(c * 16, 16)]
                ms[c] = jnp.maximum(ms[c], jnp.abs(v))
        return tuple(ms)

    return lax.fori_loop(0, iters, body, carry)


_R1 = _N1 // _DIM
_R2 = _N2 // _DIM
_R3 = _N3 // _DIM
_CR3 = _R3 // 3
_R3T = 2640
_CR3T = _R3T // 6


def _sc_scales(w1, w2, w3):
    @functools.partial(
        _mesh,
        out_type=jax.ShapeDtypeStruct((_NW, 12, 16), jnp.float32),
        scratch_types=[
            pltpu.VMEM((_CR3, _DIM), jnp.float32),
            pltpu.VMEM((_CR3, _DIM), jnp.float32),
            pltpu.VMEM((12, 16), jnp.float32),
            pltpu.SemaphoreType.DMA,
        ],
    )
    def k(w1_hbm, w2_hbm, w3_hbm, out_hbm, buf0, buf1, acc, sem):
        tid = _tile_id()
        zero4 = (jnp.zeros(16, jnp.float32),) * 4
        bufs = (buf0, buf1)

        phases = [(w1_hbm, tid * _R1, _R1, 7, 0),
                  (w2_hbm, tid * _R2, _R2, 7, 1)]
        phases += [(w3_hbm, tid * _R3 + kk * _CR3, _CR3, 13, 2)
                   for kk in range(3)]
        n = len(phases)

        def start(i):
            src, off, rows, _, _ = phases[i]
            return pltpu.async_copy(src.at[pl.ds(off, rows)],
                                    bufs[i % 2].at[pl.ds(0, rows)], sem)

        ms = [zero4, zero4, zero4]
        h = [None] * n
        h[0] = start(0)
        for i in range(n):
            if i + 1 < n:
                h[i + 1] = start(i + 1)
            h[i].wait()
            _, _, rows, unroll, slot = phases[i]
            ms[slot] = _reduce_chunk(bufs[i % 2], rows, unroll, ms[slot])

        for c in range(4):
            acc[0 + c, :] = ms[0][c]
            acc[4 + c, :] = ms[1][c]
            acc[8 + c, :] = ms[2][c]
        pltpu.sync_copy(acc, out_hbm.at[tid])

    return k(w1, w2, w3)


def _quant_chunk(buf, nrows, unroll_rows, pre, sc):
    iters = nrows // unroll_rows
    assert iters * unroll_rows == nrows

    def body(it, carry):
        base = it * unroll_rows
        for j in range(unroll_rows):
            for c in range(4):
                v = buf[base + j, pl.ds(c * 16, 16)]
                r = (v * pre[c] + _MAGIC) - _MAGIC
                buf[base + j, pl.ds(c * 16, 16)] = r * sc
        return carry

    lax.fori_loop(0, iters, body, 0)


def _scale_chunk(buf, nrows, svecs):
    def body(it, carry):
        for c in range(4):
            v = buf[it, pl.ds(c * 16, 16)]
            buf[it, pl.ds(c * 16, 16)] = v * svecs[c]
        return carry

    lax.fori_loop(0, nrows, body, 0)


def _sc_table(partials, w0, w1, w2, w3, s0, s1, s2, s3):
    @functools.partial(
        _mesh,
        out_type=jax.ShapeDtypeStruct((_VOCAB, _DIM), jnp.float32),
        scratch_types=[
            pltpu.VMEM((_CR3, _DIM), jnp.float32),
            pltpu.VMEM((_CR3, _DIM), jnp.float32),
            pltpu.VMEM((_NW, 12, 16), jnp.float32),
            pltpu.VMEM((4, 64), jnp.float32),
            pltpu.SemaphoreType.DMA,
            pltpu.SemaphoreType.DMA,
        ],
    )
    def k(p_hbm, w0, w1, w2, w3, s0, s1, s2, s3, out_hbm,
          buf0, buf1, pbuf, sbuf, sem_i, sem_o):
        tid = _tile_id()
        bufs = (buf0, buf1)

        r0 = 256 // _NW
        pre_h = [
            pltpu.async_copy(w0.at[pl.ds(tid * r0, r0)],
                             bufs[0].at[pl.ds(0, r0)], sem_i),
            pltpu.async_copy(w1.at[pl.ds(tid * _R1, _R1)],
                             bufs[1].at[pl.ds(0, _R1)], sem_i),
        ]
        pltpu.sync_copy(p_hbm, pbuf)
        for i, s in enumerate((s0, s1, s2, s3)):
            pltpu.sync_copy(s.at[0], sbuf.at[i])

        svecs = [[sbuf[t, pl.ds(c * 16, 16)] for c in range(4)]
                 for t in range(4)]
        scs = []
        for t in range(3):
            m = [pbuf[0, t * 4 + c, :] for c in range(4)]
            for i in range(1, _NW):
                for c in range(4):
                    m[c] = jnp.maximum(m[c], pbuf[i, t * 4 + c, :])
            mm = jnp.maximum(jnp.maximum(m[0] * jnp.abs(svecs[t + 1][0]),
                                         m[1] * jnp.abs(svecs[t + 1][1])),
                             jnp.maximum(m[2] * jnp.abs(svecs[t + 1][2]),
                                         m[3] * jnp.abs(svecs[t + 1][3])))
            mx = mm[0]
            for i in range(1, 16):
                mx = jnp.maximum(mx, mm[i])
            mb = jnp.zeros((16,), jnp.float32) + jnp.maximum(mx, 1e-8)
            sc = mb / _MV[t]
            inv = 1.0 / sc
            scs.append(([svecs[t + 1][c] * inv for c in range(4)], sc))

        phases = [(w0, tid * r0, r0, 0, tid * r0, 0),
                  (w1, tid * _R1, _R1, 7, 256 + tid * _R1, 1),
                  (w2, tid * _R2, _R2, 7, 2048 + tid * _R2, 2)]
        phases += [(w3, tid * _R3 + kk * _CR3, _CR3, 13,
                    16384 + tid * _R3 + kk * _CR3, 3) for kk in range(3)]
        n = len(phases)

        def start_in(i):
            src, off, rows = phases[i][:3]
            return pltpu.async_copy(src.at[pl.ds(off, rows)],
                                    bufs[i % 2].at[pl.ds(0, rows)], sem_i)

        h_in = [None] * n
        h_out = [None] * n
        h_in[0], h_in[1] = pre_h
        for i in range(n):
            if i + 1 < n:
                if i >= 1:
                    h_out[i - 1].wait()
                if i + 1 >= 2:
                    h_in[i + 1] = start_in(i + 1)
            h_in[i].wait()
            _, _, rows, unroll, out_off, tier = phases[i]
            if tier == 0:
                _scale_chunk(bufs[i % 2], rows, svecs[0])
            else:
                _quant_chunk(bufs[i % 2], rows, unroll, *scs[tier - 1])
            h_out[i] = pltpu.async_copy(
                bufs[i % 2].at[pl.ds(0, rows)],
                out_hbm.at[pl.ds(out_off, rows)], sem_o)
        h_out[n - 2].wait()
        h_out[n - 1].wait()

    return k(partials, w0, w1, w2, w3, s0, s1, s2, s3)


def _sc_gather(table, idx):
    B = idx.shape[0]
    b_per_w = B // _NW
    C = 800
    n_chunks = b_per_w // C

    @functools.partial(
        _mesh,
        out_type=jax.ShapeDtypeStruct((B, _DIM), jnp.float32),
        scratch_types=[
            pltpu.VMEM((n_chunks, C), jnp.int32),
            pltpu.VMEM((C, _DIM), jnp.float32),
            pltpu.VMEM((C, _DIM), jnp.float32),
            pltpu.SemaphoreType.DMA,
            pltpu.SemaphoreType.DMA,
            pltpu.SemaphoreType.DMA,
        ],
    )
    def k(table_hbm, idx_hbm, out_hbm, idx_v, rows0, rows1, sem_i, sem_g, sem_s):
        base = _tile_id() * b_per_w
        rows = (rows0, rows1)

        ih = [pltpu.async_copy(idx_hbm.at[pl.ds(base + c * C, C)],
                               idx_v.at[c], sem_i)
              for c in range(n_chunks)]
        for h in ih:
            h.wait()

        gh = [None] * n_chunks
        sh = [None] * n_chunks
        gh[0] = pltpu.async_copy(table_hbm.at[idx_v.at[0]], rows[0], sem_g)
        for c in range(n_chunks):
            if c + 1 < n_chunks:
                if c >= 1:
                    sh[c - 1].wait()
                gh[c + 1] = pltpu.async_copy(
                    table_hbm.at[idx_v.at[c + 1]], rows[(c + 1) % 2], sem_g)
            gh[c].wait()
            sh[c] = pltpu.async_copy(
                rows[c % 2], out_hbm.at[pl.ds(base + c * C, C)], sem_s)
        sh[n_chunks - 2].wait()
        sh[n_chunks - 1].wait()

    return k(table, idx)


def kernel(input_ids, w0, w1, w2, w3, s0, s1, s2, s3):
    partials = _sc_scales(w1, w2, w3)
    table = _sc_table(partials, w0, w1, w2, w3, s0, s1, s2, s3)
    idx = input_ids.reshape(-1).astype(jnp.int32)
    out = _sc_gather(table, idx)
    return out.reshape(input_ids.shape + (_DIM,))
